# baseline (speedup 1.0000x reference)
import functools

import jax
import jax.numpy as jnp
from jax import lax
from jax.experimental import pallas as pl
from jax.experimental.pallas import tpu as pltpu
from jax.experimental.pallas import tpu_sc as plsc

_BATCH = 16384
_SUB = 32
_DIM = 128
_NC = 2
_NS = 16
_NW = _NC * _NS
_BPW = _BATCH // _NW

_NITEMS = 1000000
_CW = 1024
_ALIGNED = (_NITEMS // _CW) * _CW
_NCHUNKS = _ALIGNED // _CW
_NTAIL = _NITEMS - _ALIGNED
_TAILN = 768
_TAIL0 = _NITEMS - _TAILN


def _build_scan():
    mesh = plsc.VectorSubcoreMesh(core_axis_name="c", subcore_axis_name="s")

    @functools.partial(
        pl.kernel,
        mesh=mesh,
        out_type=jax.ShapeDtypeStruct((_BATCH, _DIM), jnp.float32),
        compiler_params=pltpu.CompilerParams(needs_layout_passes=False),
        scratch_types=[
            pltpu.VMEM((_BATCH,), jnp.int32),
            pltpu.VMEM((_BATCH + 16,), jnp.int32),
            pltpu.VMEM((32, _CW), jnp.float32),
            pltpu.VMEM((32, _CW), jnp.float32),
            pltpu.VMEM((128, _DIM), jnp.float32),
            pltpu.VMEM((128,), jnp.int32),
            pltpu.SemaphoreType.DMA,
            pltpu.SemaphoreType.DMA,
            pltpu.SemaphoreType.DMA,
        ],
    )
    def k(ids_h, tt_h, tail_h, out_h, idx_v, mpos_v, bufa_v, bufb_v, rows_v,
          pos_v, sema, semb, sem):
        wid = lax.axis_index("s") * _NC + lax.axis_index("c")
        pltpu.sync_copy(ids_h, idx_v)

        is_last = wid == _NW - 1
        n_chunks = 31 - jnp.where(wid >= 16, 1, 0)
        lo = _CW * (30 * wid + jnp.minimum(wid, 16))
        hi = lo + n_chunks * _CW
        member_hi = hi + jnp.where(is_last, _NTAIL, 0)
        lane = lax.iota(jnp.int32, 16)

        def member(j, wcount):
            ids = idx_v[pl.ds(j * 16, 16)]
            m = (ids >= lo) & (ids < member_hi)
            pos = j * 16 + lane
            plsc.store_compressed(mpos_v.at[pl.ds(wcount, 16)], pos, mask=m)
            pc = plsc.all_reduce_population_count(m)
            return wcount + pc[0]

        wcount = lax.fori_loop(0, _BATCH // 16, member, 0)
        n_mv = (wcount + 15) // 16

        def reset_pos():
            neg = jnp.full((16,), -1, jnp.int32)
            for j in range(8):
                pos_v[pl.ds(j * 16, 16)] = neg

        def flush():
            pltpu.async_copy(
                rows_v,
                out_h.at[plsc.Indices(pos_v, ignored_value=-1)],
                sem,
            ).wait()
            reset_pos()

        def process(buf, lo_bound, hi_bound, off_base, maxoff, from_tail, k0):
            def do_members(v, k):
                pos_m = mpos_v[pl.ds(v * 16, 16)]
                valid = (v * 16 + lane) < wcount
                ids_m = plsc.load_gather(idx_v, [pos_m & (_BATCH - 1)])
                inch = valid & (ids_m >= lo_bound) & (ids_m < hi_bound)
                pcv = plsc.all_reduce_population_count(inch)[0]

                def hit(kk):
                    kk = lax.cond(kk + pcv > 128,
                                  lambda _: (flush(), 0)[1],
                                  lambda _: kk, 0)
                    off = jnp.minimum(jnp.maximum(ids_m - off_base, 0),
                                      maxoff)
                    slot = kk + plsc.cumsum(inch.astype(jnp.int32)) - 1
                    for c in range(_SUB):
                        cvec = jnp.full((16,), c, jnp.int32)
                        if from_tail:
                            f = off * _SUB + c
                            val = plsc.load_gather(
                                buf, [f >> 10, f & (_CW - 1)], mask=inch)
                        else:
                            val = plsc.load_gather(buf, [cvec, off],
                                                   mask=inch)
                        plsc.store_scatter(rows_v, [slot, cvec], val,
                                           mask=inch)
                    plsc.store_scatter(pos_v, [slot], pos_m, mask=inch)
                    return kk + pcv

                return lax.cond(pcv > 0, hit, lambda kk: kk, k)

            return lax.fori_loop(0, n_mv, do_members, k0)

        def start(ch, buf, bsem):
            nbase = pl.multiple_of(lo + ch * _CW, _CW)
            pltpu.async_copy(tt_h.at[:, pl.ds(nbase, _CW)], buf, bsem)

        def drain(buf, bsem):
            pltpu.make_async_copy(tt_h.at[:, pl.ds(0, _CW)], buf, bsem).wait()

        start(0, bufa_v, sema)
        reset_pos()

        def do_chunk(ch, k):
            base = lo + ch * _CW

            def even(kk):
                drain(bufa_v, sema)

                @pl.when(ch + 1 < n_chunks)
                def _pre():
                    start(ch + 1, bufb_v, semb)

                return process(bufa_v, base, base + _CW, base, _CW - 1,
                               False, kk)

            def odd(kk):
                drain(bufb_v, semb)

                @pl.when(ch + 1 < n_chunks)
                def _pre():
                    start(ch + 1, bufa_v, sema)

                return process(bufb_v, base, base + _CW, base, _CW - 1,
                               False, kk)

            return lax.cond((ch & 1) == 0, even, odd, k)

        k = lax.fori_loop(0, n_chunks, do_chunk, 0)

        def tail(kk):
            pltpu.sync_copy(tail_h, bufa_v.at[pl.ds(0, _TAILN * _SUB // _CW), :])
            return process(bufa_v, _ALIGNED, _NITEMS, _TAIL0, _TAILN - 1,
                           True, kk)

        k = lax.cond(is_last, tail, lambda kk: kk, k)
        flush()

    return k


def _build_assemble():
    mesh = plsc.VectorSubcoreMesh(core_axis_name="c", subcore_axis_name="s")

    @functools.partial(
        pl.kernel,
        mesh=mesh,
        out_type=jax.ShapeDtypeStruct((_BATCH, _DIM), jnp.float32),
        compiler_params=pltpu.CompilerParams(use_tc_tiling_on_sc=False),
        scratch_types=[
            pltpu.VMEM((_BPW,), jnp.int32),
            pltpu.VMEM((_BPW,), jnp.int32),
            pltpu.VMEM((_BPW,), jnp.int32),
            pltpu.VMEM((_BPW, _SUB), jnp.float32),
            pltpu.VMEM((_BPW, _SUB), jnp.float32),
            pltpu.VMEM((_BPW, _SUB), jnp.float32),
            pltpu.VMEM((_BPW, _SUB), jnp.float32),
            pltpu.SemaphoreType.DMA,
            pltpu.SemaphoreType.DMA,
            pltpu.SemaphoreType.DMA,
            pltpu.SemaphoreType.DMA,
        ],
    )
    def k(store_h, dept_h, cat_h, oi_h, st_t, dp_t, ct_t,
          out_h, i1, i2, i3, bi, r1, r2, r3,
          sb, s1, s2, s3):
        wid = lax.axis_index("s") * _NC + lax.axis_index("c")
        base = wid * _BPW
        pltpu.sync_copy(store_h.at[pl.ds(base, _BPW)], i1)
        pltpu.sync_copy(dept_h.at[pl.ds(base, _BPW)], i2)
        pltpu.sync_copy(cat_h.at[pl.ds(base, _BPW)], i3)
        cb = pltpu.async_copy(
            oi_h.at[pl.ds(base, _BPW), pl.ds(0, _SUB)], bi, sb)
        c1 = pltpu.async_copy(st_t.at[i1], r1, s1)
        c2 = pltpu.async_copy(dp_t.at[i2], r2, s2)
        c3 = pltpu.async_copy(ct_t.at[i3], r3, s3)
        cb.wait()
        pltpu.sync_copy(bi, out_h.at[pl.ds(base, _BPW), pl.ds(0, _SUB)])
        c1.wait()
        pltpu.sync_copy(r1, out_h.at[pl.ds(base, _BPW), pl.ds(1 * _SUB, _SUB)])
        c2.wait()
        pltpu.sync_copy(r2, out_h.at[pl.ds(base, _BPW), pl.ds(2 * _SUB, _SUB)])
        c3.wait()
        pltpu.sync_copy(r3, out_h.at[pl.ds(base, _BPW), pl.ds(3 * _SUB, _SUB)])

    return k


_scan = _build_scan()
_assemble = _build_assemble()


def kernel(item_ids, store_ids, dept_ids, cat_ids,
           item_table, store_table, dept_table, cat_table):
    item_t = item_table.T
    tail = jnp.reshape(
        lax.slice(item_table, (_TAIL0, 0), (_NITEMS, _SUB)),
        (_TAILN * _SUB // _CW, _CW))
    out_item = _scan(item_ids, item_t, tail)
    return _assemble(store_ids, dept_ids, cat_ids, out_item,
                     store_table, dept_table, cat_table)


# speedup vs baseline: 1.0119x; 1.0119x over previous
import functools

import jax
import jax.numpy as jnp
from jax import lax
from jax.experimental import pallas as pl
from jax.experimental.pallas import tpu as pltpu
from jax.experimental.pallas import tpu_sc as plsc

_BATCH = 16384
_SUB = 32
_DIM = 128
_NC = 2
_NS = 16
_NW = _NC * _NS
_BPW = _BATCH // _NW

_NITEMS = 1000000
_CW = 1024
_ALIGNED = (_NITEMS // _CW) * _CW
_NCHUNKS = _ALIGNED // _CW
_NTAIL = _NITEMS - _ALIGNED
_TAILN = 768
_TAIL0 = _NITEMS - _TAILN


def _build_scan():
    mesh = plsc.VectorSubcoreMesh(core_axis_name="c", subcore_axis_name="s")

    @functools.partial(
        pl.kernel,
        mesh=mesh,
        out_type=jax.ShapeDtypeStruct((_BATCH, _DIM), jnp.float32),
        compiler_params=pltpu.CompilerParams(needs_layout_passes=False),
        scratch_types=[
            pltpu.VMEM((_BATCH,), jnp.int32),
            pltpu.VMEM((_BATCH + 16,), jnp.int32),
            pltpu.VMEM((32, _CW), jnp.float32),
            pltpu.VMEM((32, _CW), jnp.float32),
            pltpu.VMEM((128, _DIM), jnp.float32),
            pltpu.VMEM((128,), jnp.int32),
            pltpu.SemaphoreType.DMA,
            pltpu.SemaphoreType.DMA,
            pltpu.SemaphoreType.DMA,
        ],
    )
    def k(ids_h, tt_h, tail_h, out_h, idx_v, mpos_v, bufa_v, bufb_v, rows_v,
          pos_v, sema, semb, sem):
        wid = lax.axis_index("s") * _NC + lax.axis_index("c")
        pltpu.sync_copy(ids_h, idx_v)

        is_last = wid == _NW - 1
        n_chunks = 31 - jnp.where(wid >= 16, 1, 0)
        lo = _CW * (30 * wid + jnp.minimum(wid, 16))
        hi = lo + n_chunks * _CW
        member_hi = hi + jnp.where(is_last, _NTAIL, 0)
        lane = lax.iota(jnp.int32, 16)

        def start(ch, buf, bsem):
            nbase = pl.multiple_of(lo + ch * _CW, _CW)
            pltpu.async_copy(tt_h.at[:, pl.ds(nbase, _CW)], buf, bsem)

        def drain(buf, bsem):
            pltpu.make_async_copy(tt_h.at[:, pl.ds(0, _CW)], buf, bsem).wait()

        start(0, bufa_v, sema)
        start(1, bufb_v, semb)

        def member(j, wcount):
            ids = idx_v[pl.ds(j * 16, 16)]
            m = (ids >= lo) & (ids < member_hi)
            pos = j * 16 + lane
            plsc.store_compressed(mpos_v.at[pl.ds(wcount, 16)], pos, mask=m)
            pc = plsc.all_reduce_population_count(m)
            return wcount + pc[0]

        wcount = lax.fori_loop(0, _BATCH // 16, member, 0)
        n_mv = (wcount + 15) // 16

        def reset_pos():
            neg = jnp.full((16,), -1, jnp.int32)
            for j in range(8):
                pos_v[pl.ds(j * 16, 16)] = neg

        def flush():
            pltpu.async_copy(
                rows_v,
                out_h.at[plsc.Indices(pos_v, ignored_value=-1)],
                sem,
            ).wait()
            reset_pos()

        def process(buf, lo_bound, hi_bound, off_base, maxoff, from_tail, k0):
            def do_members(v, k):
                pos_m = mpos_v[pl.ds(v * 16, 16)]
                valid = (v * 16 + lane) < wcount
                ids_m = plsc.load_gather(idx_v, [pos_m & (_BATCH - 1)])
                inch = valid & (ids_m >= lo_bound) & (ids_m < hi_bound)
                pcv = plsc.all_reduce_population_count(inch)[0]

                def hit(kk):
                    kk = lax.cond(kk + pcv > 128,
                                  lambda _: (flush(), 0)[1],
                                  lambda _: kk, 0)
                    off = jnp.minimum(jnp.maximum(ids_m - off_base, 0),
                                      maxoff)
                    slot = kk + plsc.cumsum(inch.astype(jnp.int32)) - 1
                    for c in range(_SUB):
                        cvec = jnp.full((16,), c, jnp.int32)
                        if from_tail:
                            f = off * _SUB + c
                            val = plsc.load_gather(
                                buf, [f >> 10, f & (_CW - 1)], mask=inch)
                        else:
                            val = plsc.load_gather(buf, [cvec, off],
                                                   mask=inch)
                        plsc.store_scatter(rows_v, [slot, cvec], val,
                                           mask=inch)
                    plsc.store_scatter(pos_v, [slot], pos_m, mask=inch)
                    return kk + pcv

                return lax.cond(pcv > 0, hit, lambda kk: kk, k)

            return lax.fori_loop(0, n_mv, do_members, k0)

        reset_pos()

        def do_chunk(ch, k):
            base = lo + ch * _CW

            def even(kk):
                drain(bufa_v, sema)
                kk = process(bufa_v, base, base + _CW, base, _CW - 1,
                             False, kk)

                @pl.when(ch + 2 < n_chunks)
                def _pre():
                    start(ch + 2, bufa_v, sema)

                return kk

            def odd(kk):
                drain(bufb_v, semb)
                kk = process(bufb_v, base, base + _CW, base, _CW - 1,
                             False, kk)

                @pl.when(ch + 2 < n_chunks)
                def _pre():
                    start(ch + 2, bufb_v, semb)

                return kk

            return lax.cond((ch & 1) == 0, even, odd, k)

        k = lax.fori_loop(0, n_chunks, do_chunk, 0)

        def tail(kk):
            pltpu.sync_copy(tail_h, bufa_v.at[pl.ds(0, _TAILN * _SUB // _CW), :])
            return process(bufa_v, _ALIGNED, _NITEMS, _TAIL0, _TAILN - 1,
                           True, kk)

        k = lax.cond(is_last, tail, lambda kk: kk, k)
        flush()

    return k


def _build_assemble():
    mesh = plsc.VectorSubcoreMesh(core_axis_name="c", subcore_axis_name="s")

    @functools.partial(
        pl.kernel,
        mesh=mesh,
        out_type=jax.ShapeDtypeStruct((_BATCH, _DIM), jnp.float32),
        compiler_params=pltpu.CompilerParams(use_tc_tiling_on_sc=False),
        scratch_types=[
            pltpu.VMEM((_BPW,), jnp.int32),
            pltpu.VMEM((_BPW,), jnp.int32),
            pltpu.VMEM((_BPW,), jnp.int32),
            pltpu.VMEM((_BPW, _SUB), jnp.float32),
            pltpu.VMEM((_BPW, _SUB), jnp.float32),
            pltpu.VMEM((_BPW, _SUB), jnp.float32),
            pltpu.VMEM((_BPW, _SUB), jnp.float32),
            pltpu.SemaphoreType.DMA,
            pltpu.SemaphoreType.DMA,
            pltpu.SemaphoreType.DMA,
            pltpu.SemaphoreType.DMA,
        ],
    )
    def k(store_h, dept_h, cat_h, oi_h, st_t, dp_t, ct_t,
          out_h, i1, i2, i3, bi, r1, r2, r3,
          sb, s1, s2, s3):
        wid = lax.axis_index("s") * _NC + lax.axis_index("c")
        base = wid * _BPW
        pltpu.sync_copy(store_h.at[pl.ds(base, _BPW)], i1)
        pltpu.sync_copy(dept_h.at[pl.ds(base, _BPW)], i2)
        pltpu.sync_copy(cat_h.at[pl.ds(base, _BPW)], i3)
        cb = pltpu.async_copy(
            oi_h.at[pl.ds(base, _BPW), pl.ds(0, _SUB)], bi, sb)
        c1 = pltpu.async_copy(st_t.at[i1], r1, s1)
        c2 = pltpu.async_copy(dp_t.at[i2], r2, s2)
        c3 = pltpu.async_copy(ct_t.at[i3], r3, s3)
        cb.wait()
        pltpu.sync_copy(bi, out_h.at[pl.ds(base, _BPW), pl.ds(0, _SUB)])
        c1.wait()
        pltpu.sync_copy(r1, out_h.at[pl.ds(base, _BPW), pl.ds(1 * _SUB, _SUB)])
        c2.wait()
        pltpu.sync_copy(r2, out_h.at[pl.ds(base, _BPW), pl.ds(2 * _SUB, _SUB)])
        c3.wait()
        pltpu.sync_copy(r3, out_h.at[pl.ds(base, _BPW), pl.ds(3 * _SUB, _SUB)])

    return k


_scan = _build_scan()
_assemble = _build_assemble()


def kernel(item_ids, store_ids, dept_ids, cat_ids,
           item_table, store_table, dept_table, cat_table):
    item_t = item_table.T
    tail = jnp.reshape(
        lax.slice(item_table, (_TAIL0, 0), (_NITEMS, _SUB)),
        (_TAILN * _SUB // _CW, _CW))
    out_item = _scan(item_ids, item_t, tail)
    return _assemble(store_ids, dept_ids, cat_ids, out_item,
                     store_table, dept_table, cat_table)
